# baseline (speedup 1.0000x reference)
import jax
import jax.numpy as jnp
from jax import lax
from jax.experimental import pallas as pl
from jax.experimental.pallas import tpu as pltpu
from jax.experimental.pallas import tpu_sc as plsc

_MAX_K = 8
_TAU = 0.7
_BLOCK_T = 2048
_N_CHUNKS = 1
_NW = 32
_L = 16

_S8 = [(0, 1), (2, 3), (4, 5), (6, 7),
       (0, 2), (1, 3), (4, 6), (5, 7),
       (1, 2), (5, 6),
       (0, 4), (1, 5), (2, 6), (3, 7),
       (2, 4), (3, 5),
       (1, 2), (3, 4), (5, 6)]
_M8 = [(0, 4), (1, 5), (2, 6), (3, 7),
       (0, 2), (1, 3), (4, 6), (5, 7),
       (0, 1), (2, 3), (4, 5), (6, 7)]


def _mm_softmax_body(x_ref, w_ref, p_ref):
    logits = jax.lax.dot_general(
        x_ref[...], w_ref[...], (((1,), (0,)), ((), ())),
        preferred_element_type=jnp.float32)
    mx = jnp.max(logits, axis=-1, keepdims=True)
    ex = jnp.exp(logits - mx)
    p_ref[...] = ex / jnp.sum(ex, axis=-1, keepdims=True)


def _router_probs(x, w):
    t, d = x.shape
    e = w.shape[1]
    return pl.pallas_call(
        _mm_softmax_body,
        grid=(t // _BLOCK_T,),
        in_specs=[
            pl.BlockSpec((_BLOCK_T, d), lambda i: (i, 0)),
            pl.BlockSpec((d, e), lambda i: (0, 0)),
        ],
        out_specs=pl.BlockSpec((_BLOCK_T, e), lambda i: (i, 0)),
        out_shape=jax.ShapeDtypeStruct((t, e), jnp.float32),
    )(x, w)


def _ce_desc(v, i, j):
    a, b = v[i], v[j]
    v[i] = jnp.maximum(a, b)
    v[j] = jnp.minimum(a, b)


def _top8_desc(p):
    runs = []
    for grp in range(8):
        v = [p[grp * 8 + i] for i in range(8)]
        for (i, j) in _S8:
            _ce_desc(v, i, j)
        runs.append(v)
    while len(runs) > 1:
        nxt = []
        for a, b in zip(runs[::2], runs[1::2]):
            t = [jnp.maximum(a[i], b[7 - i]) for i in range(8)]
            for (i, j) in _M8:
                _ce_desc(t, i, j)
            nxt.append(t)
        runs = nxt
    return runs[0]


def _make_gate_body(rows_per, e):
    def _gate_body(probs_hbm, out_hbm, in_v, out_v):
        wid = lax.axis_index("s") * 2 + lax.axis_index("c")
        base = wid * rows_per * e
        pltpu.sync_copy(probs_hbm.at[pl.ds(base, rows_per * e)], in_v)

        @plsc.parallel_loop(0, rows_per, _L, unroll=4)
        def group(row0):
            fidx = (lax.iota(jnp.int32, _L) + row0) * e
            p = [plsc.load_gather(in_v, [fidx + c]) for c in range(e)]
            m = _top8_desc(p)
            cum = m[0]
            denom = m[0]
            kcnt = jnp.ones((_L,), jnp.int32)
            cutoff = m[0]
            for k in range(1, _MAX_K):
                cum = cum + m[k]
                keep = (cum - m[k]) < _TAU
                denom = denom + jnp.where(keep, m[k], 0.0)
                kcnt = kcnt + jnp.where(keep, 1, 0)
                cutoff = jnp.where(keep, m[k], cutoff)
            invd = 1.0 / (denom + 1e-9)
            gcnt = jnp.where(m[0] > cutoff, 1, 0)
            for k in range(1, _MAX_K):
                gcnt = gcnt + jnp.where(m[k] > cutoff, 1, 0)
            rcnt = kcnt - gcnt
            acc = jnp.zeros((_L,), jnp.int32)
            for c in range(e):
                gt = p[c] > cutoff
                eq = p[c] == cutoff
                kp = gt | (eq & (acc < rcnt))
                val = jnp.where(kp, p[c] * invd, 0.0)
                plsc.store_scatter(out_v, [fidx + c], val)
                acc = acc + jnp.where(eq, 1, 0)

        del group
        pltpu.sync_copy(out_v, out_hbm.at[pl.ds(base, rows_per * e)])

    return _gate_body


def _gate_sc(probs):
    t, e = probs.shape
    rows_per = t // _NW
    mesh = plsc.VectorSubcoreMesh(core_axis_name="c", subcore_axis_name="s")
    f = pl.kernel(
        _make_gate_body(rows_per, e),
        out_type=jax.ShapeDtypeStruct((t * e,), jnp.float32),
        mesh=mesh,
        compiler_params=pltpu.CompilerParams(needs_layout_passes=False),
        scratch_types=[
            pltpu.VMEM((rows_per * e,), jnp.float32),
            pltpu.VMEM((rows_per * e,), jnp.float32),
        ],
    )
    return f(probs.reshape(-1)).reshape(t, e)


@jax.jit
def kernel(x, w_gating):
    t = x.shape[0]
    tc = t // _N_CHUNKS
    outs = []
    for i in range(_N_CHUNKS):
        probs = _router_probs(jax.lax.slice_in_dim(x, i * tc, (i + 1) * tc), w_gating)
        outs.append(_gate_sc(probs))
    return jnp.concatenate(outs, axis=0)


# speedup vs baseline: 1.0016x; 1.0016x over previous
import jax
import jax.numpy as jnp
from jax import lax
from jax.experimental import pallas as pl
from jax.experimental.pallas import tpu as pltpu
from jax.experimental.pallas import tpu_sc as plsc

_MAX_K = 8
_TAU = 0.7
_BLOCK_T = 1024
_N_CHUNKS = 1
_NW = 32
_L = 16

_S8 = [(0, 1), (2, 3), (4, 5), (6, 7),
       (0, 2), (1, 3), (4, 6), (5, 7),
       (1, 2), (5, 6),
       (0, 4), (1, 5), (2, 6), (3, 7),
       (2, 4), (3, 5),
       (1, 2), (3, 4), (5, 6)]
_M8 = [(0, 4), (1, 5), (2, 6), (3, 7),
       (0, 2), (1, 3), (4, 6), (5, 7),
       (0, 1), (2, 3), (4, 5), (6, 7)]


def _mm_softmax_body(x_ref, w_ref, p_ref):
    logits = jax.lax.dot_general(
        x_ref[...], w_ref[...], (((1,), (0,)), ((), ())),
        preferred_element_type=jnp.float32)
    mx = jnp.max(logits, axis=-1, keepdims=True)
    ex = jnp.exp(logits - mx)
    p_ref[...] = ex / jnp.sum(ex, axis=-1, keepdims=True)


def _router_probs(x, w):
    t, d = x.shape
    e = w.shape[1]
    return pl.pallas_call(
        _mm_softmax_body,
        grid=(t // _BLOCK_T,),
        in_specs=[
            pl.BlockSpec((_BLOCK_T, d), lambda i: (i, 0)),
            pl.BlockSpec((d, e), lambda i: (0, 0)),
        ],
        out_specs=pl.BlockSpec((_BLOCK_T, e), lambda i: (i, 0)),
        out_shape=jax.ShapeDtypeStruct((t, e), jnp.float32),
    )(x, w)


def _ce_desc(v, i, j):
    a, b = v[i], v[j]
    v[i] = jnp.maximum(a, b)
    v[j] = jnp.minimum(a, b)


def _top8_desc(p):
    runs = []
    for grp in range(8):
        v = [p[grp * 8 + i] for i in range(8)]
        for (i, j) in _S8:
            _ce_desc(v, i, j)
        runs.append(v)
    while len(runs) > 1:
        nxt = []
        for a, b in zip(runs[::2], runs[1::2]):
            t = [jnp.maximum(a[i], b[7 - i]) for i in range(8)]
            for (i, j) in _M8:
                _ce_desc(t, i, j)
            nxt.append(t)
        runs = nxt
    return runs[0]


def _make_gate_body(rows_per, e):
    def _gate_body(probs_hbm, out_hbm, in_v, out_v):
        wid = lax.axis_index("s") * 2 + lax.axis_index("c")
        base = wid * rows_per * e
        pltpu.sync_copy(probs_hbm.at[pl.ds(base, rows_per * e)], in_v)

        @plsc.parallel_loop(0, rows_per, _L, unroll=4)
        def group(row0):
            fidx = (lax.iota(jnp.int32, _L) + row0) * e
            p = [plsc.load_gather(in_v, [fidx + c]) for c in range(e)]
            m = _top8_desc(p)
            cum = m[0]
            denom = m[0]
            kcnt = jnp.ones((_L,), jnp.int32)
            cutoff = m[0]
            for k in range(1, _MAX_K):
                cum = cum + m[k]
                keep = (cum - m[k]) < _TAU
                denom = denom + jnp.where(keep, m[k], 0.0)
                kcnt = kcnt + jnp.where(keep, 1, 0)
                cutoff = jnp.where(keep, m[k], cutoff)
            invd = 1.0 / (denom + 1e-9)
            gcnt = jnp.where(m[0] > cutoff, 1, 0)
            for k in range(1, _MAX_K):
                gcnt = gcnt + jnp.where(m[k] > cutoff, 1, 0)
            rcnt = kcnt - gcnt
            acc = jnp.zeros((_L,), jnp.int32)
            for c in range(e):
                gt = p[c] > cutoff
                eq = p[c] == cutoff
                kp = gt | (eq & (acc < rcnt))
                val = jnp.where(kp, p[c] * invd, 0.0)
                plsc.store_scatter(out_v, [fidx + c], val)
                acc = acc + jnp.where(eq, 1, 0)

        del group
        pltpu.sync_copy(out_v, out_hbm.at[pl.ds(base, rows_per * e)])

    return _gate_body


def _gate_sc(probs):
    t, e = probs.shape
    rows_per = t // _NW
    mesh = plsc.VectorSubcoreMesh(core_axis_name="c", subcore_axis_name="s")
    f = pl.kernel(
        _make_gate_body(rows_per, e),
        out_type=jax.ShapeDtypeStruct((t * e,), jnp.float32),
        mesh=mesh,
        compiler_params=pltpu.CompilerParams(needs_layout_passes=False),
        scratch_types=[
            pltpu.VMEM((rows_per * e,), jnp.float32),
            pltpu.VMEM((rows_per * e,), jnp.float32),
        ],
    )
    return f(probs.reshape(-1)).reshape(t, e)


@jax.jit
def kernel(x, w_gating):
    t = x.shape[0]
    tc = t // _N_CHUNKS
    outs = []
    for i in range(_N_CHUNKS):
        probs = _router_probs(jax.lax.slice_in_dim(x, i * tc, (i + 1) * tc), w_gating)
        outs.append(_gate_sc(probs))
    return jnp.concatenate(outs, axis=0)


# speedup vs baseline: 1.0806x; 1.0789x over previous
import jax
import jax.numpy as jnp
from jax import lax
from jax.experimental import pallas as pl
from jax.experimental.pallas import tpu as pltpu
from jax.experimental.pallas import tpu_sc as plsc

_MAX_K = 8
_TAU = 0.7
_BLOCK_T = 1024
_N_CHUNKS = 1
_NW = 32
_L = 16

_S8 = [(0, 1), (2, 3), (4, 5), (6, 7),
       (0, 2), (1, 3), (4, 6), (5, 7),
       (1, 2), (5, 6),
       (0, 4), (1, 5), (2, 6), (3, 7),
       (2, 4), (3, 5),
       (1, 2), (3, 4), (5, 6)]
_M8 = [(0, 4), (1, 5), (2, 6), (3, 7),
       (0, 2), (1, 3), (4, 6), (5, 7),
       (0, 1), (2, 3), (4, 5), (6, 7)]


def _mm_softmax_body(x_ref, w_ref, p_ref):
    logits = jax.lax.dot_general(
        x_ref[...], w_ref[...], (((1,), (0,)), ((), ())),
        preferred_element_type=jnp.float32)
    mx = jnp.max(logits, axis=-1, keepdims=True)
    ex = jnp.exp(logits - mx)
    p_ref[...] = ex / jnp.sum(ex, axis=-1, keepdims=True)


def _router_probs(x, w):
    t, d = x.shape
    e = w.shape[1]
    return pl.pallas_call(
        _mm_softmax_body,
        grid=(t // _BLOCK_T,),
        in_specs=[
            pl.BlockSpec((_BLOCK_T, d), lambda i: (i, 0)),
            pl.BlockSpec((d, e), lambda i: (0, 0)),
        ],
        out_specs=pl.BlockSpec((_BLOCK_T, e), lambda i: (i, 0)),
        out_shape=jax.ShapeDtypeStruct((t, e), jnp.float32),
    )(x, w)


def _ce_desc(v, i, j):
    a, b = v[i], v[j]
    v[i] = jnp.maximum(a, b)
    v[j] = jnp.minimum(a, b)


def _top8_desc(p):
    runs = []
    for grp in range(8):
        v = [p[grp * 8 + i] for i in range(8)]
        for (i, j) in _S8:
            _ce_desc(v, i, j)
        runs.append(v)
    while len(runs) > 1:
        nxt = []
        for a, b in zip(runs[::2], runs[1::2]):
            t = [jnp.maximum(a[i], b[7 - i]) for i in range(8)]
            for (i, j) in _M8:
                _ce_desc(t, i, j)
            nxt.append(t)
        runs = nxt
    return runs[0]


def _make_gate_body(rows_per, e):
    def _gate_body(probs_hbm, out_hbm, in_v, out_v):
        wid = lax.axis_index("s") * 2 + lax.axis_index("c")
        base = wid * rows_per * e
        pltpu.sync_copy(probs_hbm.at[pl.ds(base, rows_per * e)], in_v)

        @plsc.parallel_loop(0, rows_per, _L, unroll=2)
        def group(row0):
            fidx = (lax.iota(jnp.int32, _L) + row0) * e
            p = [plsc.load_gather(in_v, [fidx + c]) for c in range(e)]
            m = _top8_desc(p)
            cum = m[0]
            denom = m[0]
            kcnt = jnp.ones((_L,), jnp.int32)
            cutoff = m[0]
            for k in range(1, _MAX_K):
                cum = cum + m[k]
                keep = (cum - m[k]) < _TAU
                denom = denom + jnp.where(keep, m[k], 0.0)
                kcnt = kcnt + jnp.where(keep, 1, 0)
                cutoff = jnp.where(keep, m[k], cutoff)
            invd = 1.0 / (denom + 1e-9)
            gcnt = jnp.where(m[0] > cutoff, 1, 0)
            for k in range(1, _MAX_K):
                gcnt = gcnt + jnp.where(m[k] > cutoff, 1, 0)
            rcnt = kcnt - gcnt
            acc = jnp.zeros((_L,), jnp.int32)
            for c in range(e):
                gt = p[c] > cutoff
                eq = p[c] == cutoff
                kp = gt | (eq & (acc < rcnt))
                val = jnp.where(kp, p[c] * invd, 0.0)
                plsc.store_scatter(out_v, [fidx + c], val)
                acc = acc + jnp.where(eq, 1, 0)

        del group
        pltpu.sync_copy(out_v, out_hbm.at[pl.ds(base, rows_per * e)])

    return _gate_body


def _gate_sc(probs):
    t, e = probs.shape
    rows_per = t // _NW
    mesh = plsc.VectorSubcoreMesh(core_axis_name="c", subcore_axis_name="s")
    f = pl.kernel(
        _make_gate_body(rows_per, e),
        out_type=jax.ShapeDtypeStruct((t * e,), jnp.float32),
        mesh=mesh,
        compiler_params=pltpu.CompilerParams(needs_layout_passes=False),
        scratch_types=[
            pltpu.VMEM((rows_per * e,), jnp.float32),
            pltpu.VMEM((rows_per * e,), jnp.float32),
        ],
    )
    return f(probs.reshape(-1)).reshape(t, e)


@jax.jit
def kernel(x, w_gating):
    t = x.shape[0]
    tc = t // _N_CHUNKS
    outs = []
    for i in range(_N_CHUNKS):
        probs = _router_probs(jax.lax.slice_in_dim(x, i * tc, (i + 1) * tc), w_gating)
        outs.append(_gate_sc(probs))
    return jnp.concatenate(outs, axis=0)
